# baseline (speedup 1.0000x reference)
import functools

import jax
import jax.numpy as jnp
from jax import lax
from jax.experimental import pallas as pl
from jax.experimental.pallas import tpu as pltpu
from jax.experimental.pallas import tpu_sc as plsc

B = 4
S = 2048
D = 1024
NW = 32
SPW = S // NW
CHUNK = 16
PIECES = SPW // CHUNK
STEPS = B * PIECES
NBUF = 4
LANES = 16

_mesh = plsc.VectorSubcoreMesh(core_axis_name="c", subcore_axis_name="s")


@functools.partial(
    pl.kernel,
    out_type=jax.ShapeDtypeStruct((B * S, D), jnp.float32),
    mesh=_mesh,
    scratch_types=[
        pltpu.VMEM((B, SPW), jnp.int32),
        pltpu.VMEM((2, CHUNK, D), jnp.float32),
        *[pltpu.VMEM((CHUNK, D), jnp.float32) for _ in range(NBUF)],
        *[pltpu.SemaphoreType.DMA for _ in range(NBUF)],
        *[pltpu.SemaphoreType.DMA for _ in range(NBUF)],
        pltpu.SemaphoreType.DMA,
        pltpu.SemaphoreType.DMA,
    ],
)
def _emb_kernel(x_hbm, tok_hbm, pos_hbm, out_hbm, idx_v, pos_v, *rest):
    bufs = rest[:NBUF]
    gsems = rest[NBUF:2 * NBUF]
    wsems = rest[2 * NBUF:3 * NBUF]
    psems = rest[3 * NBUF:]

    cid = lax.axis_index("c")
    sid = lax.axis_index("s")
    wid = sid * 2 + cid

    for b in range(B):
        pltpu.sync_copy(x_hbm.at[b, pl.ds(wid * SPW, SPW)], idx_v.at[b])

    def pos_load(piece):
        k = piece % 2
        return pltpu.async_copy(
            pos_hbm.at[pl.ds(wid * SPW + piece * CHUNK, CHUNK)],
            pos_v.at[k], psems[k])

    def gather(t):
        piece, b = divmod(t, B)
        idx = idx_v.at[b, pl.ds(piece * CHUNK, CHUNK)]
        p = t % NBUF
        return pltpu.async_copy(tok_hbm.at[idx], bufs[p], gsems[p])

    pd = [None, None]
    pd[0] = pos_load(0)
    if PIECES > 1:
        pd[1] = pos_load(1)
    gd = [None] * NBUF
    wd = [None] * NBUF
    for t in range(NBUF - 1):
        gd[t] = gather(t)

    for t in range(STEPS):
        p = t % NBUF
        piece, b = divmod(t, B)
        tn = t + NBUF - 1
        if tn < STEPS:
            pn = tn % NBUF
            if wd[pn] is not None:
                wd[pn].wait()
                wd[pn] = None
            gd[pn] = gather(tn)
        gd[p].wait()
        if b == 0 and pd[piece % 2] is not None:
            pd[piece % 2].wait()
            pd[piece % 2] = None
        buf = bufs[p]
        posb = pos_v.at[piece % 2]

        @plsc.parallel_loop(0, CHUNK)
        def _add(i):
            for j in range(D // LANES):
                sl = pl.ds(j * LANES, LANES)
                plsc.addupdate(buf.at[i, sl], posb[i, sl])

        if b == B - 1 and piece + 2 < PIECES:
            pd[piece % 2] = pos_load(piece + 2)

        row_base = b * S + wid * SPW + piece * CHUNK
        wd[p] = pltpu.async_copy(buf, out_hbm.at[pl.ds(row_base, CHUNK)], wsems[p])

    for d in wd:
        if d is not None:
            d.wait()


def kernel(x, token_table, pos_table):
    out = _emb_kernel(x.astype(jnp.int32), token_table, pos_table)
    return out.reshape(B, S, D)


# speedup vs baseline: 1.6012x; 1.6012x over previous
import functools

import jax
import jax.numpy as jnp
from jax import lax
from jax.experimental import pallas as pl
from jax.experimental.pallas import tpu as pltpu
from jax.experimental.pallas import tpu_sc as plsc

B = 4
S = 2048
D = 1024
NW = 32
SPW = S // NW
CHUNK = 32
HALF = CHUNK // 2
PIECES = SPW // CHUNK
STEPS = B * PIECES
LANES = 16

_mesh = plsc.VectorSubcoreMesh(core_axis_name="c", subcore_axis_name="s")


@functools.partial(
    pl.kernel,
    out_type=jax.ShapeDtypeStruct((B * S, D), jnp.float32),
    mesh=_mesh,
    scratch_types=[
        pltpu.VMEM((B, SPW), jnp.int32),
        pltpu.VMEM((CHUNK, D), jnp.float32),
        pltpu.VMEM((CHUNK, D), jnp.float32),
        pltpu.VMEM((CHUNK, D), jnp.float32),
        pltpu.SemaphoreType.DMA,
        pltpu.SemaphoreType.DMA,
        pltpu.SemaphoreType.DMA,
        pltpu.SemaphoreType.DMA,
        pltpu.SemaphoreType.DMA,
    ],
)
def _emb_kernel(x_hbm, tok_hbm, pos_hbm, out_hbm, idx_v, pos_v,
                buf0, buf1, g0, g1, w0, w1, psem):
    cid = lax.axis_index("c")
    sid = lax.axis_index("s")
    wid = sid * 2 + cid

    bufs = (buf0, buf1)
    gsems = (g0, g1)
    wsems = (w0, w1)

    def pos_piece_load(piece):
        return pltpu.async_copy(
            pos_hbm.at[pl.ds(wid * SPW + piece * CHUNK, CHUNK)], pos_v, psem)

    def idx_slice(t):
        piece, b = divmod(t, B)
        return idx_v.at[b, pl.ds(piece * CHUNK, CHUNK)]

    def gather(t):
        p = t % 2
        return pltpu.async_copy(tok_hbm.at[idx_slice(t)], bufs[p], gsems[p])

    idx_d = [
        pltpu.async_copy(x_hbm.at[b, pl.ds(wid * SPW, SPW)], idx_v.at[b], psem)
        for b in range(B)
    ]
    pd = pos_piece_load(0)
    idx_d[0].wait()
    gd = [None, None]
    gd[0] = gather(0)
    idx_d[1].wait()
    gd[1] = gather(1)
    idx_d[2].wait()
    idx_d[3].wait()

    wd = [None, None]
    for t in range(STEPS):
        p = t % 2
        q = 1 - p
        piece, b = divmod(t, B)
        if t + 1 < STEPS:
            for d in wd[q] or ():
                d.wait()
            wd[q] = None
            gd[q] = gather(t + 1)
        gd[p].wait()
        if b == 0:
            pd.wait()
        buf = bufs[p]
        row_base = b * S + wid * SPW + piece * CHUNK

        halves = []
        for h in range(2):
            lo = h * HALF

            @plsc.parallel_loop(0, HALF)
            def _add(i, _lo=lo):
                for j in range(D // LANES):
                    sl = pl.ds(j * LANES, LANES)
                    plsc.addupdate(buf.at[_lo + i, sl], pos_v[_lo + i, sl])

            halves.append(pltpu.async_copy(
                buf.at[pl.ds(lo, HALF)],
                out_hbm.at[pl.ds(row_base + lo, HALF)], wsems[p]))
        wd[p] = halves

        if b == B - 1 and piece + 1 < PIECES:
            pd = pos_piece_load(piece + 1)

    for half in wd:
        for d in half or ():
            d.wait()


def kernel(x, token_table, pos_table):
    out = _emb_kernel(x.astype(jnp.int32), token_table, pos_table)
    return out.reshape(B, S, D)
